# baseline (speedup 1.0000x reference)
import jax
import jax.numpy as jnp
from jax.experimental import pallas as pl


def _add_body(x_ref, e_ref, o_ref):
    o_ref[...] = x_ref[...] + e_ref[...]


def kernel(x, emb):
    B, T, H = x.shape
    bt = 512
    bb = 2

    return pl.pallas_call(
        _add_body,
        grid=(T // bt, B // bb),
        in_specs=[
            pl.BlockSpec((bb, bt, H), lambda t, b: (b, t, 0)),
            pl.BlockSpec((bt, H), lambda t, b: (t, 0)),
        ],
        out_specs=pl.BlockSpec((bb, bt, H), lambda t, b: (b, t, 0)),
        out_shape=jax.ShapeDtypeStruct(x.shape, x.dtype),
    )(x, emb[:T])
